# Initial kernel scaffold; baseline (speedup 1.0000x reference)
import jax
import jax.numpy as jnp
from jax.experimental import pallas as pl


def kernel(inputs, embedding):
    raise NotImplementedError("write your pallas kernel here")


# speedup vs baseline: 1.1025x; 1.1025x over previous
import functools

import jax
import jax.numpy as jnp
from jax import lax
from jax.experimental import pallas as pl
from jax.experimental.pallas import tpu as pltpu
from jax.experimental.pallas import tpu_sc as plsc

_BATCH = 16384
_HIST = 50
_F = 32
_B = _BATCH * _HIST

_NC = 2
_NS = 16
_NW = _NC * _NS
_BPW = _B // _NW
_CHUNK = 1024
_NCHUNK = _BPW // _CHUNK


def _make_gather():
    mesh = plsc.VectorSubcoreMesh(core_axis_name="c", subcore_axis_name="s")

    @functools.partial(
        pl.kernel,
        mesh=mesh,
        out_type=jax.ShapeDtypeStruct((_B, _F), jnp.float32),
        scratch_types=[
            pltpu.VMEM((_BPW,), jnp.int32),
            pltpu.VMEM((_CHUNK, _F), jnp.float32),
            pltpu.SemaphoreType.DMA,
        ],
        compiler_params=pltpu.CompilerParams(use_tc_tiling_on_sc=False),
    )
    def gather_kernel(table_hbm, idx_hbm, out_hbm, idx_v, rows_v, sem):
        wid = lax.axis_index("s") * _NC + lax.axis_index("c")
        base = wid * _BPW
        pltpu.sync_copy(idx_hbm.at[pl.ds(base, _BPW)], idx_v)

        def body(j, carry):
            off = j * _CHUNK
            pltpu.async_copy(
                table_hbm.at[idx_v.at[pl.ds(off, _CHUNK)]], rows_v, sem
            ).wait()
            pltpu.sync_copy(rows_v, out_hbm.at[pl.ds(base + off, _CHUNK)])
            return carry

        lax.fori_loop(0, _NCHUNK, body, 0)

    return gather_kernel


_gather = _make_gather()


def kernel(inputs, embedding):
    idx = inputs.reshape(-1).astype(jnp.int32)
    out = _gather(embedding, idx)
    return out.reshape(inputs.shape + (_F,))


# speedup vs baseline: 1.1090x; 1.0059x over previous
import functools

import jax
import jax.numpy as jnp
from jax import lax
from jax.experimental import pallas as pl
from jax.experimental.pallas import tpu as pltpu
from jax.experimental.pallas import tpu_sc as plsc

_BATCH = 16384
_HIST = 50
_F = 32
_B = _BATCH * _HIST

_NC = 2
_NS = 16
_NW = _NC * _NS
_BPW = _B // _NW
_CHUNK = 640
_NBUF = 4
_NCHUNK = _BPW // _CHUNK
_NGROUP = _NCHUNK // _NBUF


def _make_gather():
    mesh = plsc.VectorSubcoreMesh(core_axis_name="c", subcore_axis_name="s")

    @functools.partial(
        pl.kernel,
        mesh=mesh,
        out_type=jax.ShapeDtypeStruct((_B, _F), jnp.float32),
        scratch_types=[
            pltpu.VMEM((_BPW,), jnp.int32),
            pltpu.VMEM((_NBUF, _CHUNK, _F), jnp.float32),
            pltpu.SemaphoreType.DMA((_NBUF,)),
            pltpu.SemaphoreType.DMA((_NBUF,)),
        ],
        compiler_params=pltpu.CompilerParams(use_tc_tiling_on_sc=False),
    )
    def gather_kernel(table_hbm, idx_hbm, out_hbm, idx_v, rows_v, gsem, wsem):
        wid = lax.axis_index("s") * _NC + lax.axis_index("c")
        base = wid * _BPW
        pltpu.sync_copy(idx_hbm.at[pl.ds(base, _BPW)], idx_v)

        def group(g, carry):
            off0 = g * (_NBUF * _CHUNK)
            for b in range(_NBUF):
                off = off0 + b * _CHUNK

                @pl.when(g > 0)
                def _drain(b=b):
                    pltpu.make_async_copy(
                        rows_v.at[b], out_hbm.at[pl.ds(base, _CHUNK)], wsem.at[b]
                    ).wait()

                pltpu.async_copy(
                    table_hbm.at[idx_v.at[pl.ds(off, _CHUNK)]],
                    rows_v.at[b],
                    gsem.at[b],
                )
            for b in range(_NBUF):
                off = off0 + b * _CHUNK
                pltpu.make_async_copy(
                    table_hbm.at[idx_v.at[pl.ds(off, _CHUNK)]],
                    rows_v.at[b],
                    gsem.at[b],
                ).wait()
                pltpu.async_copy(
                    rows_v.at[b], out_hbm.at[pl.ds(base + off, _CHUNK)], wsem.at[b]
                )
            return carry

        lax.fori_loop(0, _NGROUP, group, 0)
        for b in range(_NBUF):
            pltpu.make_async_copy(
                rows_v.at[b], out_hbm.at[pl.ds(base, _CHUNK)], wsem.at[b]
            ).wait()

    return gather_kernel


_gather = _make_gather()


def kernel(inputs, embedding):
    idx = inputs.reshape(-1).astype(jnp.int32)
    out = _gather(embedding, idx)
    return out.reshape(inputs.shape + (_F,))


# speedup vs baseline: 1.7943x; 1.6180x over previous
import functools

import jax
import jax.numpy as jnp
from jax import lax
from jax.experimental import pallas as pl
from jax.experimental.pallas import tpu as pltpu
from jax.experimental.pallas import tpu_sc as plsc

_BATCH = 16384
_HIST = 50
_F = 32

_NC = 2
_NS = 16
_NW = _NC * _NS
_EPW = _BATCH // _NW
_CB = 16
_NBUF = 4
_NCHUNK = _EPW // _CB
_NGROUP = _NCHUNK // _NBUF


def _make_gather():
    mesh = plsc.VectorSubcoreMesh(core_axis_name="c", subcore_axis_name="s")

    @functools.partial(
        pl.kernel,
        mesh=mesh,
        out_type=jax.ShapeDtypeStruct((_BATCH, _HIST, _F), jnp.float32),
        scratch_types=[
            pltpu.VMEM((_EPW, _HIST), jnp.int32),
            pltpu.VMEM((_NBUF, _CB, _HIST, _F), jnp.float32),
            pltpu.SemaphoreType.DMA((_NBUF,)),
            pltpu.SemaphoreType.DMA((_NBUF,)),
        ],
        compiler_params=pltpu.CompilerParams(use_tc_tiling_on_sc=False),
    )
    def gather_kernel(table_hbm, idx_hbm, out_hbm, idx_v, rows_v, gsem, wsem):
        wid = lax.axis_index("s") * _NC + lax.axis_index("c")
        ebase = wid * _EPW
        pltpu.sync_copy(idx_hbm.at[pl.ds(ebase, _EPW), :], idx_v)

        def group(g, carry):
            e0 = g * (_NBUF * _CB)
            for b in range(_NBUF):

                @pl.when(g > 0)
                def _drain(b=b):
                    pltpu.make_async_copy(
                        rows_v.at[b], out_hbm.at[pl.ds(ebase, _CB)], wsem.at[b]
                    ).wait()

                def fire(k, c, b=b):
                    e = e0 + b * _CB + k
                    pltpu.async_copy(
                        table_hbm.at[idx_v.at[e]], rows_v.at[b, k], gsem.at[b]
                    )
                    return c

                lax.fori_loop(0, _CB, fire, 0)
            for b in range(_NBUF):

                def drain_g(k, c, b=b):
                    e = e0 + b * _CB + k
                    pltpu.make_async_copy(
                        table_hbm.at[idx_v.at[e]], rows_v.at[b, k], gsem.at[b]
                    ).wait()
                    return c

                lax.fori_loop(0, _CB, drain_g, 0)
                pltpu.async_copy(
                    rows_v.at[b],
                    out_hbm.at[pl.ds(ebase + e0 + b * _CB, _CB)],
                    wsem.at[b],
                )
            return carry

        lax.fori_loop(0, _NGROUP, group, 0)
        for b in range(_NBUF):
            pltpu.make_async_copy(
                rows_v.at[b], out_hbm.at[pl.ds(ebase, _CB)], wsem.at[b]
            ).wait()

    return gather_kernel


_gather = _make_gather()


def kernel(inputs, embedding):
    return _gather(embedding, inputs.astype(jnp.int32))
